# baseline (speedup 1.0000x reference)
import functools

import jax
import jax.numpy as jnp
from jax import lax
from jax.experimental import pallas as pl
from jax.experimental.pallas import tpu as pltpu
from jax.experimental.pallas import tpu_sc as plsc

BATCH, SEQ, DIM = 4, 8192, 768
NC, NS = 2, 16
NW = NC * NS
ROWS_PER_W = (BATCH * SEQ) // NW
CHUNK = 64
NCHUNK = ROWS_PER_W // CHUNK
CHUNK_EL = CHUNK * DIM
VECS = CHUNK_EL // 16
UNROLL = 8

_mesh = plsc.VectorSubcoreMesh(core_axis_name="c", subcore_axis_name="s")


@functools.partial(
    pl.kernel,
    mesh=_mesh,
    out_type=jax.ShapeDtypeStruct((BATCH * SEQ * DIM,), jnp.float32),
    scratch_types=[
        pltpu.VMEM((CHUNK_EL,), jnp.float32),
        pltpu.VMEM((CHUNK_EL,), jnp.float32),
    ],
)
def _pos_add(x_hbm, pos_hbm, out_hbm, xbuf, pbuf):
    wid = lax.axis_index("s") * NC + lax.axis_index("c")
    base = wid * ROWS_PER_W * DIM
    pbase = (wid % (SEQ // ROWS_PER_W)) * ROWS_PER_W * DIM

    def chunk_body(ci, _):
        off = ci * CHUNK_EL
        pltpu.sync_copy(x_hbm.at[pl.ds(base + off, CHUNK_EL)], xbuf)
        pltpu.sync_copy(pos_hbm.at[pl.ds(pbase + off, CHUNK_EL)], pbuf)

        def add_body(i, _):
            s = i * (16 * UNROLL)
            for u in range(UNROLL):
                o = s + u * 16
                xbuf[pl.ds(o, 16)] = xbuf[pl.ds(o, 16)] + pbuf[pl.ds(o, 16)]
            return 0

        lax.fori_loop(0, VECS // UNROLL, add_body, 0)
        pltpu.sync_copy(xbuf, out_hbm.at[pl.ds(base + off, CHUNK_EL)])
        return 0

    lax.fori_loop(0, NCHUNK, chunk_body, 0)


def kernel(x, pos_emb):
    out = _pos_add(x.reshape(-1), pos_emb.reshape(-1))
    return out.reshape(x.shape)


# speedup vs baseline: 1.1225x; 1.1225x over previous
import functools

import jax
import jax.numpy as jnp
from jax import lax
from jax.experimental import pallas as pl
from jax.experimental.pallas import tpu as pltpu
from jax.experimental.pallas import tpu_sc as plsc

BATCH, SEQ, DIM = 4, 8192, 768
NC, NS = 2, 16
NW = NC * NS
S_PER_W = SEQ // NW
CHUNK = 32
PCHUNKS = S_PER_W // CHUNK
NITER = PCHUNKS * BATCH
CHUNK_EL = CHUNK * DIM
UNROLL = 8
ADD_ITERS = CHUNK_EL // (16 * UNROLL)

_mesh = plsc.VectorSubcoreMesh(core_axis_name="c", subcore_axis_name="s")


@functools.partial(
    pl.kernel,
    mesh=_mesh,
    out_type=jax.ShapeDtypeStruct((BATCH * SEQ * DIM,), jnp.float32),
    scratch_types=[
        pltpu.VMEM((2, CHUNK_EL), jnp.float32),
        pltpu.VMEM((2, CHUNK_EL), jnp.float32),
        pltpu.SemaphoreType.DMA,
        pltpu.SemaphoreType.DMA,
        pltpu.SemaphoreType.DMA,
        pltpu.SemaphoreType.DMA,
        pltpu.SemaphoreType.DMA,
        pltpu.SemaphoreType.DMA,
    ],
)
def _pos_add(x_hbm, pos_hbm, out_hbm, xbuf, pbuf,
             xsem0, xsem1, psem0, psem1, osem0, osem1):
    wid = lax.axis_index("s") * NC + lax.axis_index("c")
    pbase = wid * S_PER_W * DIM

    xsems = (xsem0, xsem1)
    psems = (psem0, psem1)
    osems = (osem0, osem1)

    def x_off(i):
        pc, b = divmod(i, BATCH)
        return b * SEQ * DIM + pbase + pc * CHUNK_EL

    def xload(i):
        return pltpu.async_copy(
            x_hbm.at[pl.ds(x_off(i), CHUNK_EL)], xbuf.at[i % 2], xsems[i % 2])

    def pload(pc):
        return pltpu.async_copy(
            pos_hbm.at[pl.ds(pbase + pc * CHUNK_EL, CHUNK_EL)],
            pbuf.at[pc % 2], psems[pc % 2])

    pload(0)
    xload(0)

    stores = [None, None]
    for i in range(NITER):
        bi = i % 2
        pc, b = divmod(i, BATCH)
        if i + 1 < NITER:
            if stores[(i + 1) % 2] is not None:
                stores[(i + 1) % 2].wait()
                stores[(i + 1) % 2] = None
            xload(i + 1)
            if (i + 1) % BATCH == 0:
                pload(pc + 1)
        pltpu.make_async_copy(
            x_hbm.at[pl.ds(x_off(i), CHUNK_EL)], xbuf.at[bi], xsems[bi]).wait()
        if b == 0:
            pltpu.make_async_copy(
                pos_hbm.at[pl.ds(pbase + pc * CHUNK_EL, CHUNK_EL)],
                pbuf.at[pc % 2], psems[pc % 2]).wait()

        xb = xbuf.at[bi]
        pb = pbuf.at[pc % 2]

        def add_body(k, _, xb=xb, pb=pb):
            s = k * (16 * UNROLL)
            for u in range(UNROLL):
                o = s + u * 16
                xb[pl.ds(o, 16)] = xb[pl.ds(o, 16)] + pb[pl.ds(o, 16)]
            return 0

        lax.fori_loop(0, ADD_ITERS, add_body, 0)

        stores[bi] = pltpu.async_copy(
            xbuf.at[bi], out_hbm.at[pl.ds(x_off(i), CHUNK_EL)], osems[bi])

    for st in stores:
        if st is not None:
            st.wait()


def kernel(x, pos_emb):
    out = _pos_add(x.reshape(-1), pos_emb.reshape(-1))
    return out.reshape(x.shape)


# speedup vs baseline: 1.2147x; 1.0822x over previous
import functools

import jax
import jax.numpy as jnp
from jax import lax
from jax.experimental import pallas as pl
from jax.experimental.pallas import tpu as pltpu
from jax.experimental.pallas import tpu_sc as plsc

BATCH, SEQ, DIM = 4, 8192, 768
NC, NS = 2, 16
NW = NC * NS
S_PER_W = SEQ // NW
CHUNK = 32
PCHUNKS = S_PER_W // CHUNK
NITER = PCHUNKS * BATCH
UNROLL = 8
VEC_ITERS = DIM // (16 * UNROLL)

_mesh = plsc.VectorSubcoreMesh(core_axis_name="c", subcore_axis_name="s")


@functools.partial(
    pl.kernel,
    mesh=_mesh,
    out_type=jax.ShapeDtypeStruct((BATCH, SEQ, DIM), jnp.float32),
    scratch_types=[
        pltpu.VMEM((2, CHUNK, DIM), jnp.float32),
        pltpu.VMEM((2, CHUNK, DIM), jnp.float32),
        pltpu.SemaphoreType.DMA,
        pltpu.SemaphoreType.DMA,
        pltpu.SemaphoreType.DMA,
        pltpu.SemaphoreType.DMA,
        pltpu.SemaphoreType.DMA,
        pltpu.SemaphoreType.DMA,
    ],
)
def _pos_add(x_hbm, pos_hbm, out_hbm, xbuf, pbuf,
             xsem0, xsem1, psem0, psem1, osem0, osem1):
    wid = lax.axis_index("s") * NC + lax.axis_index("c")
    s0 = wid * S_PER_W

    xsems = (xsem0, xsem1)
    psems = (psem0, psem1)
    osems = (osem0, osem1)

    def xload(i):
        pc, b = divmod(i, BATCH)
        return pltpu.async_copy(
            x_hbm.at[b, pl.ds(s0 + pc * CHUNK, CHUNK), :],
            xbuf.at[i % 2], xsems[i % 2])

    def pload(pc):
        return pltpu.async_copy(
            pos_hbm.at[pl.ds(s0 + pc * CHUNK, CHUNK), :],
            pbuf.at[pc % 2], psems[pc % 2])

    pwaits = [None, None]
    pwaits[0] = pload(0)
    xwaits = [None, None]
    xwaits[0] = xload(0)
    stores = [None, None]

    for i in range(NITER):
        bi = i % 2
        pc, b = divmod(i, BATCH)
        if i + 1 < NITER:
            if stores[(i + 1) % 2] is not None:
                stores[(i + 1) % 2].wait()
                stores[(i + 1) % 2] = None
            xwaits[(i + 1) % 2] = xload(i + 1)
            if (i + 1) % BATCH == 0 and pc + 1 < PCHUNKS:
                pwaits[(pc + 1) % 2] = pload(pc + 1)
        xwaits[bi].wait()
        xwaits[bi] = None
        if b == 0 and pwaits[pc % 2] is not None:
            pwaits[pc % 2].wait()
            pwaits[pc % 2] = None

        xb = xbuf.at[bi]
        pb = pbuf.at[pc % 2]

        def row_body(r, _, xb=xb, pb=pb):
            def add_body(j, _):
                base = j * (16 * UNROLL)
                for u in range(UNROLL):
                    o = base + u * 16
                    xb[r, pl.ds(o, 16)] = xb[r, pl.ds(o, 16)] + pb[r, pl.ds(o, 16)]
                return 0

            lax.fori_loop(0, VEC_ITERS, add_body, 0)
            return 0

        lax.fori_loop(0, CHUNK, row_body, 0)

        stores[bi] = pltpu.async_copy(
            xbuf.at[bi], out_hbm.at[b, pl.ds(s0 + pc * CHUNK, CHUNK), :],
            osems[bi])

    for st in stores:
        if st is not None:
            st.wait()


def kernel(x, pos_emb):
    return _pos_add(x, pos_emb)


# speedup vs baseline: 1.2284x; 1.0113x over previous
import functools

import jax
import jax.numpy as jnp
from jax import lax
from jax.experimental import pallas as pl
from jax.experimental.pallas import tpu as pltpu
from jax.experimental.pallas import tpu_sc as plsc

BATCH, SEQ, DIM = 4, 8192, 768
NC, NS = 2, 16
NW = NC * NS
S_PER_W = SEQ // NW
CHUNK = 32
PCHUNKS = S_PER_W // CHUNK
NITER = PCHUNKS * BATCH
UNROLL = 8
VEC_ITERS = DIM // (16 * UNROLL)

_mesh = plsc.VectorSubcoreMesh(core_axis_name="c", subcore_axis_name="s")


@functools.partial(
    pl.kernel,
    mesh=_mesh,
    out_type=jax.ShapeDtypeStruct((BATCH, SEQ, DIM), jnp.float32),
    scratch_types=[
        pltpu.VMEM((2, CHUNK, DIM), jnp.float32),
        pltpu.VMEM((CHUNK, DIM), jnp.float32),
        pltpu.VMEM((2, CHUNK, DIM), jnp.float32),
        pltpu.SemaphoreType.DMA,
        pltpu.SemaphoreType.DMA,
        pltpu.SemaphoreType.DMA,
        pltpu.SemaphoreType.DMA,
        pltpu.SemaphoreType.DMA,
    ],
)
def _pos_add(x_hbm, pos_hbm, out_hbm, xbuf, pbuf, obuf,
             xsem0, xsem1, psem, osem0, osem1):
    wid = lax.axis_index("s") * NC + lax.axis_index("c")
    s0 = wid * S_PER_W

    xsems = (xsem0, xsem1)
    osems = (osem0, osem1)

    def xload(i):
        pc, b = divmod(i, BATCH)
        return pltpu.async_copy(
            x_hbm.at[b, pl.ds(s0 + pc * CHUNK, CHUNK), :],
            xbuf.at[i % 2], xsems[i % 2])

    def pload(pc):
        return pltpu.async_copy(
            pos_hbm.at[pl.ds(s0 + pc * CHUNK, CHUNK), :], pbuf, psem)

    pwait = pload(0)
    xwaits = [None, None]
    xwaits[0] = xload(0)
    stores = [None, None]

    for i in range(NITER):
        bi = i % 2
        pc, b = divmod(i, BATCH)
        if i + 1 < NITER:
            xwaits[(i + 1) % 2] = xload(i + 1)
        xwaits[bi].wait()
        xwaits[bi] = None
        if b == 0 and pwait is not None:
            pwait.wait()
            pwait = None
        if stores[bi] is not None:
            stores[bi].wait()
            stores[bi] = None

        xb = xbuf.at[bi]
        ob = obuf.at[bi]

        def row_body(r, _, xb=xb, ob=ob):
            def add_body(j, _):
                base = j * (16 * UNROLL)
                for u in range(UNROLL):
                    o = base + u * 16
                    ob[r, pl.ds(o, 16)] = xb[r, pl.ds(o, 16)] + pbuf[r, pl.ds(o, 16)]
                return 0

            lax.fori_loop(0, VEC_ITERS, add_body, 0)
            return 0

        lax.fori_loop(0, CHUNK, row_body, 0)

        stores[bi] = pltpu.async_copy(
            obuf.at[bi], out_hbm.at[b, pl.ds(s0 + pc * CHUNK, CHUNK), :],
            osems[bi])

        if b == BATCH - 1 and pc + 1 < PCHUNKS:
            pwait = pload(pc + 1)

    for st in stores:
        if st is not None:
            st.wait()


def kernel(x, pos_emb):
    return _pos_add(x, pos_emb)


# speedup vs baseline: 1.3266x; 1.0799x over previous
import functools

import jax
import jax.numpy as jnp
from jax import lax
from jax.experimental import pallas as pl
from jax.experimental.pallas import tpu as pltpu
from jax.experimental.pallas import tpu_sc as plsc

BATCH, SEQ, DIM = 4, 8192, 768
NC, NS = 2, 16
NW = NC * NS
S_PER_W = SEQ // NW
CHUNK = 32
PCHUNKS = S_PER_W // CHUNK
NITER = PCHUNKS * BATCH
CHUNK_EL = CHUNK * DIM
UNROLL = 8
ADD_ITERS = CHUNK_EL // (16 * UNROLL)

_mesh = plsc.VectorSubcoreMesh(core_axis_name="c", subcore_axis_name="s")


@functools.partial(
    pl.kernel,
    mesh=_mesh,
    out_type=jax.ShapeDtypeStruct((BATCH, SEQ, DIM), jnp.float32),
    scratch_types=[
        pltpu.VMEM((2, CHUNK_EL), jnp.float32),
        pltpu.VMEM((CHUNK_EL,), jnp.float32),
        pltpu.VMEM((2, CHUNK_EL), jnp.float32),
        pltpu.SemaphoreType.DMA,
        pltpu.SemaphoreType.DMA,
        pltpu.SemaphoreType.DMA,
        pltpu.SemaphoreType.DMA,
        pltpu.SemaphoreType.DMA,
    ],
)
def _pos_add(x_hbm, pos_hbm, out_hbm, xbuf, pbuf, obuf,
             xsem0, xsem1, psem, osem0, osem1):
    wid = lax.axis_index("s") * NC + lax.axis_index("c")
    s0 = wid * S_PER_W

    xsems = (xsem0, xsem1)
    osems = (osem0, osem1)

    def rows_loop(fn):
        def body(r, _):
            fn(r)
            return 0
        lax.fori_loop(0, CHUNK, body, 0)

    def split(i):
        return i // BATCH, lax.rem(i, BATCH)

    def issue_x(i, p):
        pc, b = split(i)
        r0 = s0 + pc * CHUNK
        rows_loop(lambda r: pltpu.async_copy(
            x_hbm.at[b, r0 + r, :],
            xbuf.at[p, pl.ds(r * DIM, DIM)], xsems[p]))

    def wait_x(i, p):
        pc, b = split(i)
        r0 = s0 + pc * CHUNK
        rows_loop(lambda r: pltpu.make_async_copy(
            x_hbm.at[b, r0 + r, :],
            xbuf.at[p, pl.ds(r * DIM, DIM)], xsems[p]).wait())

    def issue_p(pc):
        r0 = s0 + pc * CHUNK
        rows_loop(lambda r: pltpu.async_copy(
            pos_hbm.at[r0 + r, :], pbuf.at[pl.ds(r * DIM, DIM)], psem))

    def wait_p(pc):
        r0 = s0 + pc * CHUNK
        rows_loop(lambda r: pltpu.make_async_copy(
            pos_hbm.at[r0 + r, :], pbuf.at[pl.ds(r * DIM, DIM)], psem).wait())

    def issue_o(i, p):
        pc, b = split(i)
        r0 = s0 + pc * CHUNK
        rows_loop(lambda r: pltpu.async_copy(
            obuf.at[p, pl.ds(r * DIM, DIM)],
            out_hbm.at[b, r0 + r, :], osems[p]))

    def wait_o(i, p):
        pc, b = split(i)
        r0 = s0 + pc * CHUNK
        rows_loop(lambda r: pltpu.make_async_copy(
            obuf.at[p, pl.ds(r * DIM, DIM)],
            out_hbm.at[b, r0 + r, :], osems[p]).wait())

    issue_p(0)
    issue_x(0, 0)

    def chunk_pair(k, _):
        for p in (0, 1):
            i = 2 * k + p
            pc, b = split(i)
            if p == 0:
                issue_x(i + 1, 1)
            else:
                @pl.when(i + 1 < NITER)
                def _():
                    issue_x(i + 1, 0)
            wait_x(i, p)

            @pl.when(b == 0)
            def _():
                wait_p(pc)

            @pl.when(i >= 2)
            def _():
                wait_o(i - 2, p)

            xb = xbuf.at[p]
            ob = obuf.at[p]

            def add_body(j, _, xb=xb, ob=ob):
                base = j * (16 * UNROLL)
                for u in range(UNROLL):
                    o = base + u * 16
                    ob[pl.ds(o, 16)] = xb[pl.ds(o, 16)] + pbuf[pl.ds(o, 16)]
                return 0

            lax.fori_loop(0, ADD_ITERS, add_body, 0)

            issue_o(i, p)

            @pl.when((b == BATCH - 1) & (pc + 1 < PCHUNKS))
            def _():
                issue_p(pc + 1)
        return 0

    lax.fori_loop(0, NITER // 2, chunk_pair, 0)

    wait_o(NITER - 2, 0)
    wait_o(NITER - 1, 1)


def kernel(x, pos_emb):
    return _pos_add(x, pos_emb)


# speedup vs baseline: 2.8359x; 2.1377x over previous
import functools

import jax
import jax.numpy as jnp
from jax import lax
from jax.experimental import pallas as pl
from jax.experimental.pallas import tpu as pltpu
from jax.experimental.pallas import tpu_sc as plsc

BATCH, SEQ, DIM = 4, 8192, 768
NC, NS = 2, 16
NW = NC * NS
S_PER_W = SEQ // NW
CHUNK = 32
PCHUNKS = S_PER_W // CHUNK
NITER = PCHUNKS * BATCH
CHUNK_EL = CHUNK * DIM
UNROLL = 8
ADD_ITERS = CHUNK_EL // (16 * UNROLL)

_mesh = plsc.VectorSubcoreMesh(core_axis_name="c", subcore_axis_name="s")


@functools.partial(
    pl.kernel,
    mesh=_mesh,
    out_type=jax.ShapeDtypeStruct((BATCH, SEQ, DIM), jnp.float32),
    scratch_types=[
        pltpu.VMEM((2, CHUNK_EL), jnp.float32),
        pltpu.VMEM((CHUNK_EL,), jnp.float32),
        pltpu.VMEM((2, CHUNK_EL), jnp.float32),
        pltpu.SemaphoreType.DMA,
        pltpu.SemaphoreType.DMA,
        pltpu.SemaphoreType.DMA,
        pltpu.SemaphoreType.DMA,
        pltpu.SemaphoreType.DMA,
    ],
)
def _pos_add(x_hbm, pos_hbm, out_hbm, xbuf, pbuf, obuf,
             xsem0, xsem1, psem, osem0, osem1):
    wid = lax.axis_index("s") * NC + lax.axis_index("c")
    s0 = wid * S_PER_W

    xsems = (xsem0, xsem1)
    osems = (osem0, osem1)

    def rows_loop(fn):
        def body(r, _):
            fn(r)
            return 0
        lax.fori_loop(0, CHUNK, body, 0)

    def split(i):
        return i // BATCH, lax.rem(i, BATCH)

    def issue_x(i, p):
        pc, b = split(i)
        r0 = s0 + pc * CHUNK
        rows_loop(lambda r: pltpu.async_copy(
            x_hbm.at[b, r0 + r, :],
            xbuf.at[p, pl.ds(r * DIM, DIM)], xsems[p]))

    def wait_x(i, p):
        pc, b = split(i)
        r0 = s0 + pc * CHUNK
        rows_loop(lambda r: pltpu.make_async_copy(
            x_hbm.at[b, r0 + r, :],
            xbuf.at[p, pl.ds(r * DIM, DIM)], xsems[p]).wait())

    def issue_p(pc):
        r0 = s0 + pc * CHUNK
        rows_loop(lambda r: pltpu.async_copy(
            pos_hbm.at[r0 + r, :], pbuf.at[pl.ds(r * DIM, DIM)], psem))

    def wait_p(pc):
        r0 = s0 + pc * CHUNK
        rows_loop(lambda r: pltpu.make_async_copy(
            pos_hbm.at[r0 + r, :], pbuf.at[pl.ds(r * DIM, DIM)], psem).wait())

    def issue_o(i, p):
        pc, b = split(i)
        r0 = s0 + pc * CHUNK
        rows_loop(lambda r: pltpu.async_copy(
            obuf.at[p, pl.ds(r * DIM, DIM)],
            out_hbm.at[b, r0 + r, :], osems[p]))

    def wait_o(i, p):
        pc, b = split(i)
        r0 = s0 + pc * CHUNK
        rows_loop(lambda r: pltpu.make_async_copy(
            obuf.at[p, pl.ds(r * DIM, DIM)],
            out_hbm.at[b, r0 + r, :], osems[p]).wait())

    issue_p(0)
    issue_x(0, 0)

    def chunk_pair(k, _):
        for p in (0, 1):
            i = 2 * k + p
            pc, b = split(i)
            if p == 0:
                issue_x(i + 1, 1)
            else:
                @pl.when(i + 1 < NITER)
                def _():
                    issue_x(i + 1, 0)
            wait_x(i, p)

            @pl.when(b == 0)
            def _():
                wait_p(pc)

            @pl.when(i >= 2)
            def _():
                wait_o(i - 2, p)

            xb = xbuf.at[p]
            ob = obuf.at[p]

            @plsc.parallel_loop(0, CHUNK_EL, 16, unroll=UNROLL)
            def _(o, xb=xb, ob=ob):
                ob[pl.ds(o, 16)] = xb[pl.ds(o, 16)] + pbuf[pl.ds(o, 16)]

            issue_o(i, p)

            @pl.when((b == BATCH - 1) & (pc + 1 < PCHUNKS))
            def _():
                issue_p(pc + 1)
        return 0

    lax.fori_loop(0, NITER // 2, chunk_pair, 0)

    wait_o(NITER - 2, 0)
    wait_o(NITER - 1, 1)


def kernel(x, pos_emb):
    return _pos_add(x, pos_emb)
